# baseline (speedup 1.0000x reference)
import functools

import jax
import jax.numpy as jnp
from jax import lax
from jax.experimental import pallas as pl
from jax.experimental.pallas import tpu as pltpu
from jax.experimental.pallas import tpu_sc as plsc

D = 768
B = 4
T = 4096

_info = plsc.get_sparse_core_info()
NC, NS, L = _info.num_cores, _info.num_subcores, _info.num_lanes
NW = NC * NS
PW_T = T // NW
CH = 32
NCHUNK = PW_T // CH
NSTEP = NCHUNK * B
NRING = 3
UNROLL = 8


def _emb_body(tok_hbm, x_hbm, pos_hbm, out_hbm, idx_v, rows, pos, gsem, ssem, psem, isem):
    wid = lax.axis_index("s") * NC + lax.axis_index("c")
    t0 = wid * PW_T

    icopy = [
        pltpu.async_copy(x_hbm.at[b, pl.ds(t0, PW_T)], idx_v.at[b], isem)
        for b in range(B)
    ]

    def start_gather(s, k):
        c, b = s // B, s % B
        return pltpu.async_copy(
            tok_hbm.at[idx_v.at[b, pl.ds(c * CH, CH)]], rows[k], gsem[k])

    pcopy = [None] * 2
    pcopy[0] = pltpu.async_copy(pos_hbm.at[pl.ds(t0, CH)], pos[0], psem[0])
    for c in icopy:
        c.wait()
    gcopy = [None] * NRING
    scopy = [None] * NRING
    gcopy[0] = start_gather(0, 0)
    gcopy[1] = start_gather(1, 1)

    for s in range(NSTEP):
        k = s % NRING
        c, b = s // B, s % B
        q = c % 2
        gcopy[k].wait()
        if b == 0:
            pcopy[q].wait()
            if c + 1 < NCHUNK:
                pcopy[1 - q] = pltpu.async_copy(
                    pos_hbm.at[pl.ds(t0 + (c + 1) * CH, CH)], pos[1 - q], psem[1 - q])

        def row_body(r, carry, _k=k, _q=q):
            def col_body(jj, carry2):
                for u in range(UNROLL):
                    sl = pl.ds(jj * (UNROLL * L) + u * L, L)
                    rows[_k][r, sl] = rows[_k][r, sl] + pos[_q][r, sl]
                return carry2

            lax.fori_loop(0, D // (UNROLL * L), col_body, 0)
            return carry

        lax.fori_loop(0, CH, row_body, 0)

        scopy[k] = pltpu.async_copy(
            rows[k], out_hbm.at[b, pl.ds(t0 + c * CH, CH)], ssem[k])

        g = s + 2
        if g < NSTEP:
            kg = g % NRING
            if scopy[kg] is not None:
                scopy[kg].wait()
                scopy[kg] = None
            gcopy[kg] = start_gather(g, kg)

    for k in range(NRING):
        if scopy[k] is not None:
            scopy[k].wait()


@functools.partial(
    pl.kernel,
    mesh=plsc.VectorSubcoreMesh(core_axis_name="c", subcore_axis_name="s"),
    out_type=jax.ShapeDtypeStruct((B, T, D), jnp.float32),
    scratch_types=[
        pltpu.VMEM((B, PW_T), jnp.int32),
        [pltpu.VMEM((CH, D), jnp.float32) for _ in range(NRING)],
        [pltpu.VMEM((CH, D), jnp.float32) for _ in range(2)],
        [pltpu.SemaphoreType.DMA for _ in range(NRING)],
        [pltpu.SemaphoreType.DMA for _ in range(NRING)],
        [pltpu.SemaphoreType.DMA for _ in range(2)],
        pltpu.SemaphoreType.DMA,
    ],
)
def _emb_kernel(tok_hbm, x_hbm, pos_hbm, out_hbm, idx_v, rows, pos, gsem, ssem, psem, isem):
    _emb_body(tok_hbm, x_hbm, pos_hbm, out_hbm, idx_v, rows, pos, gsem, ssem, psem, isem)


def kernel(x, tok_table, pos_table):
    return _emb_kernel(tok_table, x.astype(jnp.int32), pos_table)


# speedup vs baseline: 2.2380x; 2.2380x over previous
import functools

import jax
import jax.numpy as jnp
from jax import lax
from jax.experimental import pallas as pl
from jax.experimental.pallas import tpu as pltpu
from jax.experimental.pallas import tpu_sc as plsc

D = 768
B = 4
T = 4096

_info = plsc.get_sparse_core_info()
NC, NS, L = _info.num_cores, _info.num_subcores, _info.num_lanes
NW = NC * NS
PW_T = T // NW
CH = 32
NCHUNK = PW_T // CH
NSTEP = NCHUNK * B
NRING = 3


def _emb_body(tok_hbm, x_hbm, pos_hbm, out_hbm, idx_v, rows, pos, gsem, ssem, psem, isem):
    wid = lax.axis_index("s") * NC + lax.axis_index("c")
    t0 = wid * PW_T

    icopy = [
        pltpu.async_copy(x_hbm.at[b, pl.ds(t0, PW_T)], idx_v.at[b], isem)
        for b in range(B)
    ]

    HH = CH // 2

    def start_gather(s, k):
        c, b = s // B, s % B
        return [
            pltpu.async_copy(
                tok_hbm.at[idx_v.at[b, pl.ds(c * CH + h * HH, HH)]],
                rows[k].at[pl.ds(h * HH, HH)], gsem[k][h])
            for h in range(2)
        ]

    pcopy = [None] * 2
    pcopy[0] = pltpu.async_copy(pos_hbm.at[pl.ds(t0, CH)], pos[0], psem[0])
    for c in icopy:
        c.wait()
    gcopy = [None] * NRING
    scopy = [None] * NRING
    gcopy[0] = start_gather(0, 0)
    gcopy[1] = start_gather(1, 1)

    for s in range(NSTEP):
        k = s % NRING
        c, b = s // B, s % B
        q = c % 2
        if b == 0:
            pcopy[q].wait()
            if c + 1 < NCHUNK:
                pcopy[1 - q] = pltpu.async_copy(
                    pos_hbm.at[pl.ds(t0 + (c + 1) * CH, CH)], pos[1 - q], psem[1 - q])

        scopy[k] = []
        for h in range(2):
            gcopy[k][h].wait()

            def row_body(r, carry, _k=k, _q=q, _h=h):
                gr = _h * HH + r
                for j in range(D // L):
                    sl = pl.ds(j * L, L)
                    rows[_k][gr, sl] = rows[_k][gr, sl] + pos[_q][gr, sl]
                return carry

            lax.fori_loop(0, HH, row_body, 0)
            scopy[k].append(pltpu.async_copy(
                rows[k].at[pl.ds(h * HH, HH)],
                out_hbm.at[b, pl.ds(t0 + c * CH + h * HH, HH)], ssem[k][h]))

        g = s + 2
        if g < NSTEP:
            kg = g % NRING
            if scopy[kg] is not None:
                for sc in scopy[kg]:
                    sc.wait()
                scopy[kg] = None
            gcopy[kg] = start_gather(g, kg)

    for k in range(NRING):
        if scopy[k] is not None:
            for sc in scopy[k]:
                sc.wait()


@functools.partial(
    pl.kernel,
    mesh=plsc.VectorSubcoreMesh(core_axis_name="c", subcore_axis_name="s"),
    out_type=jax.ShapeDtypeStruct((B, T, D), jnp.float32),
    scratch_types=[
        pltpu.VMEM((B, PW_T), jnp.int32),
        [pltpu.VMEM((CH, D), jnp.float32) for _ in range(NRING)],
        [pltpu.VMEM((CH, D), jnp.float32) for _ in range(2)],
        [[pltpu.SemaphoreType.DMA for _ in range(2)] for _ in range(NRING)],
        [[pltpu.SemaphoreType.DMA for _ in range(2)] for _ in range(NRING)],
        [pltpu.SemaphoreType.DMA for _ in range(2)],
        pltpu.SemaphoreType.DMA,
    ],
)
def _emb_kernel(tok_hbm, x_hbm, pos_hbm, out_hbm, idx_v, rows, pos, gsem, ssem, psem, isem):
    _emb_body(tok_hbm, x_hbm, pos_hbm, out_hbm, idx_v, rows, pos, gsem, ssem, psem, isem)


def kernel(x, tok_table, pos_table):
    return _emb_kernel(tok_table, x.astype(jnp.int32), pos_table)


# speedup vs baseline: 2.2908x; 1.0236x over previous
import functools

import jax
import jax.numpy as jnp
from jax import lax
from jax.experimental import pallas as pl
from jax.experimental.pallas import tpu as pltpu
from jax.experimental.pallas import tpu_sc as plsc

D = 768
B = 4
T = 4096

_info = plsc.get_sparse_core_info()
NC, NS, L = _info.num_cores, _info.num_subcores, _info.num_lanes
NW = NC * NS
PW_T = T // NW
CH = 32
NCHUNK = PW_T // CH
NSTEP = NCHUNK * B
NRING = 3


def _emb_body(tok_hbm, x_hbm, pos_hbm, out_hbm, idx_v, rows, pos, gsem, ssem, psem, isem):
    wid = lax.axis_index("s") * NC + lax.axis_index("c")
    t0 = wid * PW_T

    icopy = [
        pltpu.async_copy(x_hbm.at[b, pl.ds(t0, PW_T)], idx_v.at[b], isem)
        for b in range(B)
    ]

    def start_gather(s, k):
        c, b = s // B, s % B
        return pltpu.async_copy(
            tok_hbm.at[idx_v.at[b, pl.ds(c * CH, CH)]], rows[k], gsem[k])

    pcopy = [None] * 2
    pcopy[0] = pltpu.async_copy(pos_hbm.at[pl.ds(t0, CH)], pos[0], psem[0])
    for c in icopy:
        c.wait()
    gcopy = [None] * NRING
    scopy = [None] * NRING
    gcopy[0] = start_gather(0, 0)
    gcopy[1] = start_gather(1, 1)

    for s in range(NSTEP):
        k = s % NRING
        c, b = s // B, s % B
        q = c % 2
        gcopy[k].wait()
        if b == 0:
            pcopy[q].wait()
            if c + 1 < NCHUNK:
                pcopy[1 - q] = pltpu.async_copy(
                    pos_hbm.at[pl.ds(t0 + (c + 1) * CH, CH)], pos[1 - q], psem[1 - q])

        @plsc.parallel_loop(0, CH, 1, unroll=2)
        def row_body(r, _k=k, _q=q):
            for j in range(D // L):
                sl = pl.ds(j * L, L)
                rows[_k][r, sl] = rows[_k][r, sl] + pos[_q][r, sl]

        scopy[k] = pltpu.async_copy(
            rows[k], out_hbm.at[b, pl.ds(t0 + c * CH, CH)], ssem[k])

        g = s + 2
        if g < NSTEP:
            kg = g % NRING
            if scopy[kg] is not None:
                scopy[kg].wait()
                scopy[kg] = None
            gcopy[kg] = start_gather(g, kg)

    for k in range(NRING):
        if scopy[k] is not None:
            scopy[k].wait()


@functools.partial(
    pl.kernel,
    mesh=plsc.VectorSubcoreMesh(core_axis_name="c", subcore_axis_name="s"),
    out_type=jax.ShapeDtypeStruct((B, T, D), jnp.float32),
    scratch_types=[
        pltpu.VMEM((B, PW_T), jnp.int32),
        [pltpu.VMEM((CH, D), jnp.float32) for _ in range(NRING)],
        [pltpu.VMEM((CH, D), jnp.float32) for _ in range(2)],
        [pltpu.SemaphoreType.DMA for _ in range(NRING)],
        [pltpu.SemaphoreType.DMA for _ in range(NRING)],
        [pltpu.SemaphoreType.DMA for _ in range(2)],
        pltpu.SemaphoreType.DMA,
    ],
)
def _emb_kernel(tok_hbm, x_hbm, pos_hbm, out_hbm, idx_v, rows, pos, gsem, ssem, psem, isem):
    _emb_body(tok_hbm, x_hbm, pos_hbm, out_hbm, idx_v, rows, pos, gsem, ssem, psem, isem)


def kernel(x, tok_table, pos_table):
    return _emb_kernel(tok_table, x.astype(jnp.int32), pos_table)


# speedup vs baseline: 2.3562x; 1.0286x over previous
import functools

import jax
import jax.numpy as jnp
from jax import lax
from jax.experimental import pallas as pl
from jax.experimental.pallas import tpu as pltpu
from jax.experimental.pallas import tpu_sc as plsc

D = 768
B = 4
T = 4096

_info = plsc.get_sparse_core_info()
NC, NS, L = _info.num_cores, _info.num_subcores, _info.num_lanes
NW = NC * NS
PW_T = T // NW
CH = 32
NCHUNK = PW_T // CH
NSTEP = NCHUNK * B
NRING = 3


def _emb_body(tok_hbm, x_hbm, pos_hbm, out_hbm, idx_v, rows, pos, gsem, ssem, psem, isem):
    wid = lax.axis_index("s") * NC + lax.axis_index("c")
    t0 = wid * PW_T

    icopy = [
        pltpu.async_copy(x_hbm.at[b, pl.ds(t0, PW_T)], idx_v.at[b], isem)
        for b in range(B)
    ]

    def start_gather(s, k):
        c, b = s // B, s % B
        return pltpu.async_copy(
            tok_hbm.at[idx_v.at[b, pl.ds(c * CH, CH)]], rows[k], gsem[k])

    pcopy = [None] * 2
    pcopy[0] = pltpu.async_copy(pos_hbm.at[pl.ds(t0, CH)], pos[0], psem[0])
    for c in icopy:
        c.wait()
    gcopy = [None] * NRING
    scopy = [None] * NRING
    gcopy[0] = start_gather(0, 0)
    gcopy[1] = start_gather(1, 1)

    for s in range(NSTEP):
        k = s % NRING
        c, b = s // B, s % B
        q = c % 2
        gcopy[k].wait()
        if b == 0:
            pcopy[q].wait()
            if c + 1 < NCHUNK:
                pcopy[1 - q] = pltpu.async_copy(
                    pos_hbm.at[pl.ds(t0 + (c + 1) * CH, CH)], pos[1 - q], psem[1 - q])

        @plsc.parallel_loop(0, CH, 1, unroll=1)
        def row_body(r, _k=k, _q=q):
            for j in range(D // L):
                sl = pl.ds(j * L, L)
                rows[_k][r, sl] = rows[_k][r, sl] + pos[_q][r, sl]

        scopy[k] = pltpu.async_copy(
            rows[k], out_hbm.at[b, pl.ds(t0 + c * CH, CH)], ssem[k])

        g = s + 2
        if g < NSTEP:
            kg = g % NRING
            if scopy[kg] is not None:
                scopy[kg].wait()
                scopy[kg] = None
            gcopy[kg] = start_gather(g, kg)

    for k in range(NRING):
        if scopy[k] is not None:
            scopy[k].wait()


@functools.partial(
    pl.kernel,
    mesh=plsc.VectorSubcoreMesh(core_axis_name="c", subcore_axis_name="s"),
    out_type=jax.ShapeDtypeStruct((B, T, D), jnp.float32),
    scratch_types=[
        pltpu.VMEM((B, PW_T), jnp.int32),
        [pltpu.VMEM((CH, D), jnp.float32) for _ in range(NRING)],
        [pltpu.VMEM((CH, D), jnp.float32) for _ in range(2)],
        [pltpu.SemaphoreType.DMA for _ in range(NRING)],
        [pltpu.SemaphoreType.DMA for _ in range(NRING)],
        [pltpu.SemaphoreType.DMA for _ in range(2)],
        pltpu.SemaphoreType.DMA,
    ],
)
def _emb_kernel(tok_hbm, x_hbm, pos_hbm, out_hbm, idx_v, rows, pos, gsem, ssem, psem, isem):
    _emb_body(tok_hbm, x_hbm, pos_hbm, out_hbm, idx_v, rows, pos, gsem, ssem, psem, isem)


def kernel(x, tok_table, pos_table):
    return _emb_kernel(tok_table, x.astype(jnp.int32), pos_table)


# speedup vs baseline: 2.3688x; 1.0053x over previous
import functools

import jax
import jax.numpy as jnp
from jax import lax
from jax.experimental import pallas as pl
from jax.experimental.pallas import tpu as pltpu
from jax.experimental.pallas import tpu_sc as plsc

D = 768
B = 4
T = 4096

_info = plsc.get_sparse_core_info()
NC, NS, L = _info.num_cores, _info.num_subcores, _info.num_lanes
NW = NC * NS
PW_T = T // NW
CH = 32
NCHUNK = PW_T // CH
NSTEP = NCHUNK * B
NRING = 3


def _emb_body(tok_hbm, x_hbm, pos_hbm, out_hbm, idx_v, rows, pos, gsem, ssem, psem, isem):
    wid = lax.axis_index("s") * NC + lax.axis_index("c")
    t0 = wid * PW_T

    icopy = [
        pltpu.async_copy(x_hbm.at[b, pl.ds(t0, PW_T)], idx_v.at[b], isem)
        for b in range(B)
    ]

    def start_gather(s, k):
        c, b = s // B, s % B
        return pltpu.async_copy(
            tok_hbm.at[idx_v.at[b, pl.ds(c * CH, CH)]], rows[k], gsem[k])

    pcopy = [None] * 2
    pcopy[0] = pltpu.async_copy(pos_hbm.at[pl.ds(t0, CH)], pos[0], psem[0])
    gcopy = [None] * NRING
    scopy = [None] * NRING
    for g in range(NRING):
        icopy[g].wait()
        gcopy[g] = start_gather(g, g)
    icopy[B - 1].wait()

    for s in range(NSTEP):
        k = s % NRING
        c, b = s // B, s % B
        q = c % 2
        gcopy[k].wait()
        if b == 0:
            pcopy[q].wait()
            if c + 1 < NCHUNK:
                pcopy[1 - q] = pltpu.async_copy(
                    pos_hbm.at[pl.ds(t0 + (c + 1) * CH, CH)], pos[1 - q], psem[1 - q])

        def row_body(r, carry, _k=k, _q=q):
            for j in range(D // L):
                sl = pl.ds(j * L, L)
                rows[_k][r, sl] = rows[_k][r, sl] + pos[_q][r, sl]
            return carry

        lax.fori_loop(0, CH, row_body, 0)

        scopy[k] = pltpu.async_copy(
            rows[k], out_hbm.at[b, pl.ds(t0 + c * CH, CH)], ssem[k])

        g = s + 2
        if s >= 1 and g < NSTEP:
            kg = g % NRING
            if scopy[kg] is not None:
                scopy[kg].wait()
                scopy[kg] = None
            gcopy[kg] = start_gather(g, kg)

    for k in range(NRING):
        if scopy[k] is not None:
            scopy[k].wait()


@functools.partial(
    pl.kernel,
    mesh=plsc.VectorSubcoreMesh(core_axis_name="c", subcore_axis_name="s"),
    out_type=jax.ShapeDtypeStruct((B, T, D), jnp.float32),
    scratch_types=[
        pltpu.VMEM((B, PW_T), jnp.int32),
        [pltpu.VMEM((CH, D), jnp.float32) for _ in range(NRING)],
        [pltpu.VMEM((CH, D), jnp.float32) for _ in range(2)],
        [pltpu.SemaphoreType.DMA for _ in range(NRING)],
        [pltpu.SemaphoreType.DMA for _ in range(NRING)],
        [pltpu.SemaphoreType.DMA for _ in range(2)],
        pltpu.SemaphoreType.DMA,
    ],
)
def _emb_kernel(tok_hbm, x_hbm, pos_hbm, out_hbm, idx_v, rows, pos, gsem, ssem, psem, isem):
    _emb_body(tok_hbm, x_hbm, pos_hbm, out_hbm, idx_v, rows, pos, gsem, ssem, psem, isem)


def kernel(x, tok_table, pos_table):
    return _emb_kernel(tok_table, x.astype(jnp.int32), pos_table)


# speedup vs baseline: 2.4029x; 1.0144x over previous
import functools

import jax
import jax.numpy as jnp
from jax import lax
from jax.experimental import pallas as pl
from jax.experimental.pallas import tpu as pltpu
from jax.experimental.pallas import tpu_sc as plsc

D = 768
B = 4
T = 4096

_info = plsc.get_sparse_core_info()
NC, NS, L = _info.num_cores, _info.num_subcores, _info.num_lanes
NW = NC * NS
PW_T = T // NW
CH = 32
NCHUNK = PW_T // CH
NSTEP = NCHUNK * B
NRING = 3


def _emb_body(tok_hbm, x_hbm, pos_hbm, out_hbm, idx_v, rows, pos, gsem, ssem, psem, isem):
    wid = lax.axis_index("s") * NC + lax.axis_index("c")
    t0 = wid * PW_T

    icopy = [
        pltpu.async_copy(x_hbm.at[b, pl.ds(t0, PW_T)], idx_v.at[b], isem)
        for b in range(B)
    ]

    def start_gather(s, k):
        c, b = s // B, s % B
        return pltpu.async_copy(
            tok_hbm.at[idx_v.at[b, pl.ds(c * CH, CH)]], rows[k], gsem[k])

    pcopy = [None] * 2
    pcopy[0] = pltpu.async_copy(pos_hbm.at[pl.ds(t0, CH)], pos[0], psem[0])
    for c in icopy:
        c.wait()
    gcopy = [None] * NRING
    scopy = [None] * NRING
    gcopy[0] = start_gather(0, 0)
    gcopy[1] = start_gather(1, 1)

    for s in range(NSTEP):
        k = s % NRING
        c, b = s // B, s % B
        q = c % 2
        gcopy[k].wait()
        if b == 0:
            pcopy[q].wait()
            if c + 1 < NCHUNK:
                pcopy[1 - q] = pltpu.async_copy(
                    pos_hbm.at[pl.ds(t0 + (c + 1) * CH, CH)], pos[1 - q], psem[1 - q])

        def row_body(r, carry, _k=k, _q=q):
            for j in range(D // L):
                sl = pl.ds(j * L, L)
                rows[_k][r, sl] = rows[_k][r, sl] + pos[_q][r, sl]
            return carry

        lax.fori_loop(0, CH, row_body, 0)

        scopy[k] = pltpu.async_copy(
            rows[k], out_hbm.at[b, pl.ds(t0 + c * CH, CH)], ssem[k])

        g = s + 2
        if g < NSTEP:
            kg = g % NRING
            if scopy[kg] is not None:
                scopy[kg].wait()
                scopy[kg] = None
            gcopy[kg] = start_gather(g, kg)

    for k in range(NRING):
        if scopy[k] is not None:
            scopy[k].wait()


@functools.partial(
    pl.kernel,
    mesh=plsc.VectorSubcoreMesh(core_axis_name="c", subcore_axis_name="s"),
    out_type=jax.ShapeDtypeStruct((B, T, D), jnp.float32),
    scratch_types=[
        pltpu.VMEM((B, PW_T), jnp.int32),
        [pltpu.VMEM((CH, D), jnp.float32) for _ in range(NRING)],
        [pltpu.VMEM((CH, D), jnp.float32) for _ in range(2)],
        [pltpu.SemaphoreType.DMA for _ in range(NRING)],
        [pltpu.SemaphoreType.DMA for _ in range(NRING)],
        [pltpu.SemaphoreType.DMA for _ in range(2)],
        pltpu.SemaphoreType.DMA,
    ],
)
def _emb_kernel(tok_hbm, x_hbm, pos_hbm, out_hbm, idx_v, rows, pos, gsem, ssem, psem, isem):
    _emb_body(tok_hbm, x_hbm, pos_hbm, out_hbm, idx_v, rows, pos, gsem, ssem, psem, isem)


def kernel(x, tok_table, pos_table):
    return _emb_kernel(tok_table, x.astype(jnp.int32), pos_table)


# speedup vs baseline: 2.4078x; 1.0020x over previous
import functools

import jax
import jax.numpy as jnp
from jax import lax
from jax.experimental import pallas as pl
from jax.experimental.pallas import tpu as pltpu
from jax.experimental.pallas import tpu_sc as plsc

D = 768
B = 4
T = 4096

_info = plsc.get_sparse_core_info()
NC, NS, L = _info.num_cores, _info.num_subcores, _info.num_lanes
NW = NC * NS
PW_T = T // NW
CH = 32
NCHUNK = PW_T // CH
NSTEP = NCHUNK * B
NRING = 3


def _emb_body(tok_hbm, x_hbm, pos_hbm, out_hbm, idx_v, rows, pos, gsem, ssem, psem, isem):
    wid = lax.axis_index("s") * NC + lax.axis_index("c")
    t0 = wid * PW_T

    icopy = [
        pltpu.async_copy(x_hbm.at[b, pl.ds(t0, PW_T)], idx_v.at[b], isem)
        for b in range(B)
    ]

    def start_gather(s, k):
        c, b = s // B, s % B
        return pltpu.async_copy(
            tok_hbm.at[idx_v.at[b, pl.ds(c * CH, CH)]], rows[k], gsem[k],
            priority=1)

    pcopy = [None] * 2
    pcopy[0] = pltpu.async_copy(pos_hbm.at[pl.ds(t0, CH)], pos[0], psem[0])
    for c in icopy:
        c.wait()
    gcopy = [None] * NRING
    scopy = [None] * NRING
    gcopy[0] = start_gather(0, 0)
    gcopy[1] = start_gather(1, 1)

    for s in range(NSTEP):
        k = s % NRING
        c, b = s // B, s % B
        q = c % 2
        gcopy[k].wait()
        if b == 0:
            pcopy[q].wait()
            if c + 1 < NCHUNK:
                pcopy[1 - q] = pltpu.async_copy(
                    pos_hbm.at[pl.ds(t0 + (c + 1) * CH, CH)], pos[1 - q], psem[1 - q])

        def row_body(r, carry, _k=k, _q=q):
            for j in range(D // L):
                sl = pl.ds(j * L, L)
                rows[_k][r, sl] = rows[_k][r, sl] + pos[_q][r, sl]
            return carry

        lax.fori_loop(0, CH, row_body, 0)

        scopy[k] = pltpu.async_copy(
            rows[k], out_hbm.at[b, pl.ds(t0 + c * CH, CH)], ssem[k])

        g = s + 2
        if g < NSTEP:
            kg = g % NRING
            if scopy[kg] is not None:
                scopy[kg].wait()
                scopy[kg] = None
            gcopy[kg] = start_gather(g, kg)

    for k in range(NRING):
        if scopy[k] is not None:
            scopy[k].wait()


@functools.partial(
    pl.kernel,
    mesh=plsc.VectorSubcoreMesh(core_axis_name="c", subcore_axis_name="s"),
    out_type=jax.ShapeDtypeStruct((B, T, D), jnp.float32),
    scratch_types=[
        pltpu.VMEM((B, PW_T), jnp.int32),
        [pltpu.VMEM((CH, D), jnp.float32) for _ in range(NRING)],
        [pltpu.VMEM((CH, D), jnp.float32) for _ in range(2)],
        [pltpu.SemaphoreType.DMA for _ in range(NRING)],
        [pltpu.SemaphoreType.DMA for _ in range(NRING)],
        [pltpu.SemaphoreType.DMA for _ in range(2)],
        pltpu.SemaphoreType.DMA,
    ],
)
def _emb_kernel(tok_hbm, x_hbm, pos_hbm, out_hbm, idx_v, rows, pos, gsem, ssem, psem, isem):
    _emb_body(tok_hbm, x_hbm, pos_hbm, out_hbm, idx_v, rows, pos, gsem, ssem, psem, isem)


def kernel(x, tok_table, pos_table):
    return _emb_kernel(tok_table, x.astype(jnp.int32), pos_table)
